# Initial kernel scaffold; baseline (speedup 1.0000x reference)
import jax
import jax.numpy as jnp
from jax.experimental import pallas as pl
from jax.experimental.pallas import tpu as pltpu
from jax.experimental.pallas import tpu_sc as plsc

_BATCH = 4096
_TIME = 200
_DIM = 32
_N = _BATCH * _TIME
_WINDOW = 128


def kernel(X, W):
    idx = X.reshape(1, _N)

    mesh = plsc.VectorSubcoreMesh(
        core_axis_name="core", subcore_axis_name="subcore"
    )

    @pl.kernel(
        out_type=jax.ShapeDtypeStruct((_N, _DIM), W.dtype),
        mesh=mesh,
    )
    def sc_gather(w_hbm, i_hbm, o_hbm):
        def body(i_vmem, o_vmem):
            pltpu.sync_copy(w_hbm.at[i_vmem.at[0]], o_vmem)

        pltpu.emit_pipeline(
            body,
            grid=(_N // _WINDOW,),
            in_specs=[
                pl.BlockSpec((1, _WINDOW), index_map=lambda i: (0, i))
            ],
            out_specs=[
                pl.BlockSpec((_WINDOW, _DIM), index_map=lambda i: (i, 0))
            ],
            core_axis_name=("core", "subcore"),
            dimension_semantics=(pltpu.PARALLEL,),
        )(i_hbm, o_hbm)

    out = sc_gather(W, idx)
    return out.reshape(_BATCH, _TIME, _DIM)


# speedup vs baseline: 1.4932x; 1.4932x over previous
import dataclasses
import functools

import jax
import jax.numpy as jnp
from jax import lax
from jax.experimental import pallas as pl
from jax.experimental.pallas import tpu as pltpu
from jax.experimental.pallas import tpu_sc as plsc

_BATCH = 4096
_TIME = 200
_DIM = 32
_N = _BATCH * _TIME
_NC = 2
_NS = 16
_NW = _NC * _NS
_BPW = _N // _NW
_CHUNK = 2560


def _compiler_params():
    cp = pltpu.CompilerParams()
    if "use_tc_tiling_on_sc" in pltpu.CompilerParams.__dataclass_fields__:
        cp = dataclasses.replace(cp, use_tc_tiling_on_sc=False)
    return cp


def kernel(X, W):
    idx = X.reshape(_N)
    mesh = plsc.VectorSubcoreMesh(core_axis_name="c", subcore_axis_name="s")

    @functools.partial(
        pl.kernel,
        out_type=jax.ShapeDtypeStruct((_N, _DIM), jnp.float32),
        mesh=mesh,
        scratch_types=[
            pltpu.VMEM((_CHUNK,), jnp.int32),
            pltpu.VMEM((_CHUNK, _DIM), jnp.float32),
            pltpu.SemaphoreType.DMA,
        ],
        compiler_params=_compiler_params(),
    )
    def sc_gather(w_hbm, i_hbm, o_hbm, idx_v, rows_v, sem):
        wid = lax.axis_index("s") * _NC + lax.axis_index("c")
        base = wid * _BPW

        @pl.loop(0, _BPW, step=_CHUNK)
        def _(off):
            pltpu.sync_copy(i_hbm.at[pl.ds(base + off, _CHUNK)], idx_v)
            pltpu.async_copy(w_hbm.at[idx_v], rows_v, sem).wait()
            pltpu.sync_copy(rows_v, o_hbm.at[pl.ds(base + off, _CHUNK)])

    out = sc_gather(W, idx)
    return out.reshape(_BATCH, _TIME, _DIM)


# speedup vs baseline: 1.4978x; 1.0031x over previous
import dataclasses
import functools

import jax
import jax.numpy as jnp
from jax import lax
from jax.experimental import pallas as pl
from jax.experimental.pallas import tpu as pltpu
from jax.experimental.pallas import tpu_sc as plsc

_BATCH = 4096
_TIME = 200
_DIM = 32
_N = _BATCH * _TIME
_NC = 2
_NS = 16
_NW = _NC * _NS
_BPW = _N // _NW
_CHUNK = 1280
_STEPS = _BPW // _CHUNK


def _compiler_params():
    cp = pltpu.CompilerParams()
    if "use_tc_tiling_on_sc" in pltpu.CompilerParams.__dataclass_fields__:
        cp = dataclasses.replace(cp, use_tc_tiling_on_sc=False)
    return cp


def kernel(X, W):
    idx = X.reshape(_N)
    mesh = plsc.VectorSubcoreMesh(core_axis_name="c", subcore_axis_name="s")

    @functools.partial(
        pl.kernel,
        out_type=jax.ShapeDtypeStruct((_N, _DIM), jnp.float32),
        mesh=mesh,
        scratch_types=[
            pltpu.VMEM((_BPW,), jnp.int32),
            pltpu.VMEM((_CHUNK, _DIM), jnp.float32),
            pltpu.VMEM((_CHUNK, _DIM), jnp.float32),
            pltpu.SemaphoreType.DMA,
            pltpu.SemaphoreType.DMA,
            pltpu.SemaphoreType.DMA,
            pltpu.SemaphoreType.DMA,
        ],
        compiler_params=_compiler_params(),
    )
    def sc_gather(w_hbm, i_hbm, o_hbm, idx_v, rows0, rows1, sg0, sg1, so0, so1):
        wid = lax.axis_index("s") * _NC + lax.axis_index("c")
        base = wid * _BPW

        pltpu.sync_copy(i_hbm.at[pl.ds(base, _BPW)], idx_v)

        def drain_store(sem):
            pltpu.make_async_copy(
                rows0, o_hbm.at[pl.ds(base, _CHUNK)], sem
            ).wait()

        @pl.loop(0, _STEPS, step=2)
        def _(g):
            c0 = base + g * _CHUNK
            c1 = c0 + _CHUNK

            @pl.when(g >= 2)
            def _():
                drain_store(so0)

            g0 = pltpu.async_copy(
                w_hbm.at[idx_v.at[pl.ds(g * _CHUNK, _CHUNK)]], rows0, sg0
            )

            @pl.when(g >= 2)
            def _():
                drain_store(so1)

            g1 = pltpu.async_copy(
                w_hbm.at[idx_v.at[pl.ds((g + 1) * _CHUNK, _CHUNK)]], rows1, sg1
            )

            g0.wait()
            pltpu.async_copy(rows0, o_hbm.at[pl.ds(c0, _CHUNK)], so0)
            g1.wait()
            pltpu.async_copy(rows1, o_hbm.at[pl.ds(c1, _CHUNK)], so1)

        drain_store(so0)
        drain_store(so1)

    out = sc_gather(W, idx)
    return out.reshape(_BATCH, _TIME, _DIM)
